# baseline (speedup 1.0000x reference)
import functools

import jax
import jax.numpy as jnp
from jax import lax
from jax.experimental import pallas as pl
from jax.experimental.pallas import tpu as pltpu
from jax.experimental.pallas import tpu_sc as plsc

N_TOK = 16384
D = 2048
E = 64
K = 8
BT = 2048
NCHUNK = 4
CHUNK = N_TOK // NCHUNK

NC = 2
NS = 16
NW = NC * NS
TPW = CHUNK // NW
L = 16
GRP = TPW // L
NEG = -3.0e38


def _matmul_block(x_ref, w_ref, *rest):
    logits_ref, logits_t_ref = rest[-2], rest[-1]
    x = x_ref[...]
    w = w_ref[...]
    logits_ref[...] = jax.lax.dot_general(
        x, w, (((1,), (1,)), ((), ())),
        preferred_element_type=jnp.float32,
        precision=jax.lax.Precision.DEFAULT,
    )
    logits_t_ref[...] = jax.lax.dot_general(
        w, x, (((1,), (1,)), ((), ())),
        preferred_element_type=jnp.float32,
        precision=jax.lax.Precision.DEFAULT,
    )


def _tc_logits_chunk(hidden_states, gate_weight, logits_buf, chunk):
    base_blk = chunk * (CHUNK // BT)
    in_specs = [
        pl.BlockSpec((BT, D), lambda i: (i + base_blk, 0)),
        pl.BlockSpec((E, D), lambda i: (0, 0)),
    ]
    args = [hidden_states, gate_weight]
    aliases = {}
    if logits_buf is not None:
        in_specs.append(pl.BlockSpec(memory_space=pltpu.MemorySpace.HBM))
        args.append(logits_buf)
        aliases = {2: 0}
    return pl.pallas_call(
        _matmul_block,
        grid=(CHUNK // BT,),
        in_specs=in_specs,
        out_specs=[
            pl.BlockSpec((BT, E), lambda i: (i + base_blk, 0)),
            pl.BlockSpec((E, BT), lambda i: (0, i)),
        ],
        out_shape=[
            jax.ShapeDtypeStruct((N_TOK, E), jnp.float32),
            jax.ShapeDtypeStruct((E, CHUNK), jnp.float32),
        ],
        input_output_aliases=aliases,
    )(*args)


def _sc_router_body(logits_t_hbm, idx_hbm, gates_hbm, buf, idxb, gateb):
    c = lax.axis_index("c")
    s = lax.axis_index("s")
    wid = s * NC + c
    base = wid * TPW
    pltpu.sync_copy(logits_t_hbm.at[:, pl.ds(base, TPW)], buf)

    lane = lax.broadcasted_iota(jnp.int32, (L,), 0)

    def group_body(g, _):
        t0 = g * L
        rows_k = (t0 + lane) * K

        def expert_body(e4, carry):
            vs = list(carry[:K])
            ids = list(carry[K:])
            for u in range(4):
                e = e4 * 4 + u
                val = buf[e, pl.ds(t0, L)]
                vid = jnp.full((L,), 0, jnp.int32) + e
                cc = [val > vs[j] for j in range(K)]
                nvs = [jnp.where(cc[0], val, vs[0])]
                nids = [jnp.where(cc[0], vid, ids[0])]
                for j in range(1, K):
                    nvs.append(
                        jnp.where(cc[j], jnp.where(cc[j - 1], vs[j - 1], val), vs[j]))
                    nids.append(
                        jnp.where(cc[j], jnp.where(cc[j - 1], ids[j - 1], vid), ids[j]))
                vs = nvs
                ids = nids
            return tuple(vs) + tuple(ids)

        init = tuple(jnp.full((L,), NEG, jnp.float32) for _ in range(K)) + \
               tuple(jnp.full((L,), 0, jnp.int32) for _ in range(K))
        res = lax.fori_loop(0, E // 4, expert_body, init)
        vs = res[:K]
        ids = res[K:]
        exps = [jnp.exp(v - vs[0]) for v in vs]
        tot = exps[0]
        for t in exps[1:]:
            tot = tot + t
        for j in range(K):
            plsc.store_scatter(idxb, [rows_k + j], ids[j])
            plsc.store_scatter(gateb, [rows_k + j], exps[j] / tot)
        return 0

    lax.fori_loop(0, GRP, group_body, 0)
    pltpu.sync_copy(idxb, idx_hbm.at[pl.ds(base * K, TPW * K)])
    pltpu.sync_copy(gateb, gates_hbm.at[pl.ds(base * K, TPW * K)])


def _sc_router(logits_t):
    mesh = plsc.VectorSubcoreMesh(core_axis_name="c", subcore_axis_name="s")
    return pl.kernel(
        _sc_router_body,
        out_type=[
            jax.ShapeDtypeStruct((CHUNK * K,), jnp.int32),
            jax.ShapeDtypeStruct((CHUNK * K,), jnp.float32),
        ],
        mesh=mesh,
        compiler_params=pltpu.CompilerParams(needs_layout_passes=False),
        scratch_types=[
            pltpu.VMEM((E, TPW), jnp.float32),
            pltpu.VMEM((TPW * K,), jnp.int32),
            pltpu.VMEM((TPW * K,), jnp.float32),
        ],
    )(logits_t)


@jax.jit
def kernel(hidden_states, gate_weight):
    logits = None
    idxs = []
    gatess = []
    for chunk in range(NCHUNK):
        logits, logits_t = _tc_logits_chunk(
            hidden_states, gate_weight, logits, chunk)
        idx_c, gates_c = _sc_router(logits_t)
        idxs.append(idx_c.reshape(CHUNK, K))
        gatess.append(gates_c.reshape(CHUNK, K))
    idx = jnp.concatenate(idxs, axis=0)
    gates = jnp.concatenate(gatess, axis=0)
    return (idx, gates, logits)


# speedup vs baseline: 1.1302x; 1.1302x over previous
import functools

import jax
import jax.numpy as jnp
from jax import lax
from jax.experimental import pallas as pl
from jax.experimental.pallas import tpu as pltpu
from jax.experimental.pallas import tpu_sc as plsc

N_TOK = 16384
D = 2048
E = 64
K = 8
BT = 2048
NCHUNK = 2
CHUNK = N_TOK // NCHUNK

NC = 2
NS = 16
NW = NC * NS
TPW = CHUNK // NW
L = 16
GRP = TPW // L
NEG = -3.0e38


def _matmul_block(x_ref, w_ref, *rest):
    logits_ref, logits_t_ref = rest[-2], rest[-1]
    x = x_ref[...]
    w = w_ref[...]
    logits_ref[...] = jax.lax.dot_general(
        x, w, (((1,), (1,)), ((), ())),
        preferred_element_type=jnp.float32,
        precision=jax.lax.Precision.DEFAULT,
    )
    logits_t_ref[...] = jax.lax.dot_general(
        w, x, (((1,), (1,)), ((), ())),
        preferred_element_type=jnp.float32,
        precision=jax.lax.Precision.DEFAULT,
    )


def _tc_logits_chunk(hidden_states, gate_weight, logits_buf, chunk):
    base_blk = chunk * (CHUNK // BT)
    in_specs = [
        pl.BlockSpec((BT, D), lambda i: (i + base_blk, 0)),
        pl.BlockSpec((E, D), lambda i: (0, 0)),
    ]
    args = [hidden_states, gate_weight]
    aliases = {}
    if logits_buf is not None:
        in_specs.append(pl.BlockSpec(memory_space=pltpu.MemorySpace.HBM))
        args.append(logits_buf)
        aliases = {2: 0}
    return pl.pallas_call(
        _matmul_block,
        grid=(CHUNK // BT,),
        in_specs=in_specs,
        out_specs=[
            pl.BlockSpec((BT, E), lambda i: (i + base_blk, 0)),
            pl.BlockSpec((E, BT), lambda i: (0, i)),
        ],
        out_shape=[
            jax.ShapeDtypeStruct((N_TOK, E), jnp.float32),
            jax.ShapeDtypeStruct((E, CHUNK), jnp.float32),
        ],
        input_output_aliases=aliases,
    )(*args)


def _sc_router_body(logits_t_hbm, idx_hbm, gates_hbm, buf, idxb, gateb):
    c = lax.axis_index("c")
    s = lax.axis_index("s")
    wid = s * NC + c
    base = wid * TPW
    pltpu.sync_copy(logits_t_hbm.at[:, pl.ds(base, TPW)], buf)

    lane = lax.broadcasted_iota(jnp.int32, (L,), 0)

    def group_body(g, _):
        t0 = g * L
        rows_k = (t0 + lane) * K

        def expert_body(e4, carry):
            vs = list(carry[:K])
            ids = list(carry[K:])
            for u in range(4):
                e = e4 * 4 + u
                val = buf[e, pl.ds(t0, L)]
                vid = jnp.full((L,), 0, jnp.int32) + e
                cc = [val > vs[j] for j in range(K)]
                nvs = [jnp.where(cc[0], val, vs[0])]
                nids = [jnp.where(cc[0], vid, ids[0])]
                for j in range(1, K):
                    nvs.append(
                        jnp.where(cc[j], jnp.where(cc[j - 1], vs[j - 1], val), vs[j]))
                    nids.append(
                        jnp.where(cc[j], jnp.where(cc[j - 1], ids[j - 1], vid), ids[j]))
                vs = nvs
                ids = nids
            return tuple(vs) + tuple(ids)

        init = tuple(jnp.full((L,), NEG, jnp.float32) for _ in range(K)) + \
               tuple(jnp.full((L,), 0, jnp.int32) for _ in range(K))
        res = lax.fori_loop(0, E // 4, expert_body, init)
        vs = res[:K]
        ids = res[K:]
        exps = [jnp.exp(v - vs[0]) for v in vs]
        tot = exps[0]
        for t in exps[1:]:
            tot = tot + t
        for j in range(K):
            plsc.store_scatter(idxb, [rows_k + j], ids[j])
            plsc.store_scatter(gateb, [rows_k + j], exps[j] / tot)
        return 0

    lax.fori_loop(0, GRP, group_body, 0)
    pltpu.sync_copy(idxb, idx_hbm.at[pl.ds(base * K, TPW * K)])
    pltpu.sync_copy(gateb, gates_hbm.at[pl.ds(base * K, TPW * K)])


def _sc_router(logits_t):
    mesh = plsc.VectorSubcoreMesh(core_axis_name="c", subcore_axis_name="s")
    return pl.kernel(
        _sc_router_body,
        out_type=[
            jax.ShapeDtypeStruct((CHUNK * K,), jnp.int32),
            jax.ShapeDtypeStruct((CHUNK * K,), jnp.float32),
        ],
        mesh=mesh,
        compiler_params=pltpu.CompilerParams(needs_layout_passes=False),
        scratch_types=[
            pltpu.VMEM((E, TPW), jnp.float32),
            pltpu.VMEM((TPW * K,), jnp.int32),
            pltpu.VMEM((TPW * K,), jnp.float32),
        ],
    )(logits_t)


@jax.jit
def kernel(hidden_states, gate_weight):
    logits = None
    idxs = []
    gatess = []
    for chunk in range(NCHUNK):
        logits, logits_t = _tc_logits_chunk(
            hidden_states, gate_weight, logits, chunk)
        idx_c, gates_c = _sc_router(logits_t)
        idxs.append(idx_c.reshape(CHUNK, K))
        gatess.append(gates_c.reshape(CHUNK, K))
    idx = jnp.concatenate(idxs, axis=0)
    gates = jnp.concatenate(gatess, axis=0)
    return (idx, gates, logits)
